# baseline (speedup 1.0000x reference)
import functools

import jax
import jax.numpy as jnp
from jax import lax
from jax.experimental import pallas as pl
from jax.experimental.pallas import tpu as pltpu
from jax.experimental.pallas import tpu_sc as plsc

NUM_EMB = 1000000
DIM = 32
BATCH = 16384
N_FIELDS = 26
PAD_FIELDS = 32
B_TOTAL = BATCH * PAD_FIELDS

_info = plsc.get_sparse_core_info()
NC = _info.num_cores
NS = _info.num_subcores
NW = NC * NS
B_PER_W = B_TOTAL // NW
CHUNK = 1024
N_CHUNKS = B_PER_W // CHUNK
NBUF = 3

_mesh = plsc.VectorSubcoreMesh(core_axis_name="c", subcore_axis_name="s")


@functools.partial(
    pl.kernel,
    mesh=_mesh,
    out_type=jax.ShapeDtypeStruct((B_TOTAL, DIM), jnp.float32),
    scratch_types=[
        pltpu.VMEM((B_PER_W,), jnp.int32),
        [pltpu.VMEM((CHUNK, DIM), jnp.float32) for _ in range(NBUF)],
        [pltpu.SemaphoreType.DMA for _ in range(NBUF)],
        [pltpu.SemaphoreType.DMA for _ in range(NBUF)],
    ],
    compiler_params=pltpu.CompilerParams(use_tc_tiling_on_sc=False),
)
def _emb_gather(idx_hbm, table_hbm, out_hbm, idx_v, rows, sem_g, sem_o):
    wid = lax.axis_index("s") * NC + lax.axis_index("c")
    base = wid * B_PER_W

    pltpu.sync_copy(idx_hbm.at[pl.ds(base, B_PER_W)], idx_v)

    def gather_copy(i, b):
        idx_c = idx_v.at[pl.ds(i * CHUNK, CHUNK)]
        return pltpu.make_async_copy(table_hbm.at[idx_c], rows[b], sem_g[b])

    for i in range(NBUF):
        gather_copy(i, i).start()

    for i in range(N_CHUNKS):
        b = i % NBUF
        off = base + i * CHUNK
        gather_copy(i, b).wait()
        store = pltpu.async_copy(rows[b], out_hbm.at[pl.ds(off, CHUNK)], sem_o[b])
        store.wait()
        if i + NBUF < N_CHUNKS:
            gather_copy(i + NBUF, b).start()


def kernel(indices, weight):
    idx_pad = jnp.pad(indices.astype(jnp.int32), ((0, 0), (0, PAD_FIELDS - N_FIELDS)))
    out = _emb_gather(idx_pad.reshape(-1), weight)
    return out.reshape(BATCH, PAD_FIELDS, DIM)[:, :N_FIELDS, :]
